# baseline (speedup 1.0000x reference)
import functools

import jax
import jax.numpy as jnp
from jax import lax
from jax.experimental import pallas as pl
from jax.experimental.pallas import tpu as pltpu
from jax.experimental.pallas import tpu_sc as plsc

N = 10000
E = 160000
F = 128
H = 64
TIN = 8
TOUT = 4
ALPHA = 0.5
RELAX = 0.7
BG = 2

NC = 2
NS = 16
L = 16
NW = NC * NS

NP = 10240


def _sc_gather_rows(table, idx, D):
    B = idx.shape[0]
    assert B % NW == 0
    b_per_w = B // NW
    mesh = plsc.VectorSubcoreMesh(core_axis_name="c", subcore_axis_name="s")

    @functools.partial(
        pl.kernel,
        mesh=mesh,
        out_type=jax.ShapeDtypeStruct((B, D), jnp.float32),
        scratch_types=[
            pltpu.VMEM((b_per_w,), jnp.int32),
            pltpu.VMEM((b_per_w, D), jnp.float32),
            pltpu.SemaphoreType.DMA,
        ],
    )
    def k(table_hbm, idx_hbm, out_hbm, idx_v, rows_v, sem):
        wid = lax.axis_index("s") * NC + lax.axis_index("c")
        base = wid * b_per_w
        pltpu.sync_copy(idx_hbm.at[pl.ds(base, b_per_w)], idx_v)
        pltpu.async_copy(table_hbm.at[idx_v], rows_v, sem).wait()
        pltpu.sync_copy(rows_v, out_hbm.at[pl.ds(base, b_per_w)])

    return k(table, idx)


def kernel(x, edge_index, edge_attr, W_g0, b_g0, W_g1, b_g1, W_g2, b_g2, W_ih, W_hh, b_ih, b_hh, W_h1, b_h1, W_h2, b_h2, W_h3, b_h3, W_fe1, b_fe1, W_fe2, b_fe2, W_fd1, b_fd1, W_fd2, b_fd2, dt):
    relu = jax.nn.relu
    src = edge_index[0]
    dst = edge_index[1]
    in_deg = jnp.bincount(dst, length=N)
    out_deg = jnp.bincount(src, length=N)
    bmask = (in_deg == 0) & (out_deg > 0)
    first_eidx = jax.ops.segment_min(jnp.arange(E, dtype=jnp.int32), src, num_segments=N)
    first_eidx = jnp.minimum(first_eidx, E - 1)
    down = dst[first_eidx]
    deg = (in_deg + out_deg + 1).astype(jnp.float32)

    dist = jnp.clip(edge_attr[:, 0], 1e-6, None)
    ew = 1.0 / dist
    ew_norm = jnp.clip(jax.ops.segment_sum(ew, dst, num_segments=N), 1e-6, None)[:, None]
    bcol = bmask[:, None]
    inv_dx = 1.0 / jnp.clip(dist[first_eidx], 1e-6, None)[:, None]

    down_p = jnp.pad(down, (0, NP - N))

    def gnn(hh):
        for (W, b) in ((W_g0, b_g0), (W_g1, b_g1), (W_g2, b_g2)):
            msg = hh[src] * ew[:, None]
            agg = jax.ops.segment_sum(msg, dst, num_segments=N) / ew_norm
            hh = relu((hh + agg) @ W.T + b)
        return hh

    def gru(xg, hs):
        gi = xg @ W_ih.T + b_ih
        gh = hs @ W_hh.T + b_hh
        ir, iz, inn = jnp.split(gi, 3, axis=1)
        hr, hz, hn = jnp.split(gh, 3, axis=1)
        r = jax.nn.sigmoid(ir + hr)
        z = jax.nn.sigmoid(iz + hz)
        n = jnp.tanh(inn + r * hn)
        return (1.0 - z) * n + z * hs

    def head(hs):
        hs = relu(hs @ W_h1.T + b_h1)
        hs = relu(hs @ W_h2.T + b_h2)
        return hs @ W_h3.T + b_h3

    def fuse(feat, down_feat, enc):
        diff = (feat - down_feat) * inv_dx
        z = jnp.concatenate([feat, down_feat, diff], axis=1)
        if enc:
            delta = relu(z @ W_fe1.T + b_fe1) @ W_fe2.T + b_fe2
        else:
            delta = relu(z @ W_fd1.T + b_fd1) @ W_fd2.T + b_fd2
        return jnp.where(bcol, feat + ALPHA * delta, feat)

    def physics(v):
        u = v[:, 0]
        f = (u[src] - u[dst] + edge_attr[:, 1]) / dist
        du = (jax.ops.segment_sum(f, dst, num_segments=N) - jax.ops.segment_sum(f, src, num_segments=N)) / deg
        return (u + dt * du)[:, None]

    h = jnp.zeros((N, H), jnp.float32)
    x_t = None
    for t in range(TIN):
        x_t = x[:, t, :]
        xd = _sc_gather_rows(x_t, down_p, F)[:N]
        x_t = fuse(x_t, xd, True)
        h = gru(gnn(x_t), h)
    cur = x_t[:, 0:1]
    preds = []
    for t in range(TOUT):
        g_in = jnp.broadcast_to(cur, (N, F)) if cur.shape[1] == 1 else cur
        h = gru(gnn(g_in), h)
        y = head(h)
        preds.append(y)
        nxt = fuse(y, y[down], False)
        ub_anchor = nxt
        for _ in range(BG):
            nxt = jnp.where(bcol, (1.0 - RELAX) * nxt + RELAX * ub_anchor, nxt)
            nxt = physics(nxt)
        cur = nxt
    return jnp.concatenate(preds, axis=1)


# speedup vs baseline: 3.7096x; 3.7096x over previous
import functools

import jax
import jax.numpy as jnp
from jax import lax
from jax.experimental import pallas as pl
from jax.experimental.pallas import tpu as pltpu
from jax.experimental.pallas import tpu_sc as plsc

N = 10000
E = 160000
F = 128
FE = 256
H = 64
TIN = 8
TOUT = 4
ALPHA = 0.5
RELAX = 0.7
BG = 2

NC = 2
NS = 16
LN = 16
NW = NC * NS

NP = 10240
NR = NP // 128
EW_ROWS = 8
CW_ROWS = 16
NCONS = NR // CW_ROWS
EP = 160256
EPT = EP // NW
VE = EPT // LN
RR = EP // 128
INIT_FIRST = float(1 << 22)

_MESH = plsc.VectorSubcoreMesh(core_axis_name="c", subcore_axis_name="s",
                               num_cores=NC, num_subcores=NS)
_SC_PARAMS = pltpu.CompilerParams(needs_layout_passes=False)


def _iota16():
    return lax.iota(jnp.int32, LN)


def _rc(n):
    return lax.shift_right_logical(n, 7), n & 127


def _run_masks(keys_sorted):
    i = _iota16()
    prev = jnp.take_along_axis(keys_sorted, jnp.maximum(i - 1, 0), axis=0)
    nxt = jnp.take_along_axis(keys_sorted, jnp.minimum(i + 1, LN - 1), axis=0)
    is_first = (i == 0) | (keys_sorted != prev)
    is_last = (i == LN - 1) | (keys_sorted != nxt)
    return is_first, is_last


def _seg_sums(vals, is_first):
    i = _iota16()
    cs = plsc.cumsum(vals)
    first_idx = plsc.cummax(jnp.where(is_first, i, 0))
    csx = jnp.take_along_axis(cs, jnp.maximum(first_idx - 1, 0), axis=0)
    excl = jnp.where(first_idx == 0, jnp.float32(0.0), csx)
    return cs - excl


def _seg_add_into2(loc2, keys_sorted, vals_sorted, is_first, is_last):
    seg = _seg_sums(vals_sorted, is_first)
    r, cc = _rc(keys_sorted)
    cur = plsc.load_gather(loc2, [r, cc], mask=is_last)
    plsc.store_scatter(loc2, [r, cc], cur + seg, mask=is_last)


def _zero2(ref2, nrows):
    def zb(j, _):
        ref2[lax.shift_right_logical(j, 3),
             pl.ds((j & 7) * LN, LN)] = jnp.zeros((LN,), jnp.float32)
        return 0
    lax.fori_loop(0, nrows * 8, zb, 0)


def _v2(ref2, j):
    return ref2[lax.shift_right_logical(j, 3), pl.ds((j & 7) * LN, LN)]


def _v2s(ref2, j, val):
    ref2[lax.shift_right_logical(j, 3), pl.ds((j & 7) * LN, LN)] = val


def _combine_rounds(sh, s, pubs, reduce_init, reduce_step, write_out):
    na = len(pubs)
    reduce_init()
    for g in range(2):
        @pl.when(lax.shift_right_logical(s, 3) == g)
        def _():
            for a in range(na):
                pltpu.sync_copy(pubs[a], sh.at[s & 7, a])
        plsc.subcore_barrier()

        @pl.when(s < NCONS)
        def _():
            def cb(i, _):
                reduce_step(i)
                return 0
            lax.fori_loop(0, 8, cb, 0)
        plsc.subcore_barrier()
    write_out()


@functools.cache
def _graph_static_kernel():
    out_type = (
        jax.ShapeDtypeStruct((NC, 6, NR, 128), jnp.float32),
        jax.ShapeDtypeStruct((EP,), jnp.float32),
        jax.ShapeDtypeStruct((EP,), jnp.float32),
    )
    scratch = [
        pltpu.VMEM((EPT,), jnp.int32),
        pltpu.VMEM((EPT,), jnp.int32),
        pltpu.VMEM((EPT,), jnp.float32),
        pltpu.VMEM((EPT,), jnp.float32),
        pltpu.VMEM((EPT,), jnp.float32),
        pltpu.VMEM((EPT,), jnp.float32),
        pltpu.VMEM((NR, 128), jnp.float32),
        pltpu.VMEM((NR, 128), jnp.float32),
        pltpu.VMEM((NR, 128), jnp.float32),
        pltpu.VMEM((NR, 128), jnp.float32),
        pltpu.VMEM((NR, 128), jnp.float32),
        pltpu.VMEM((NR, 128), jnp.float32),
        pltpu.VMEM((CW_ROWS, 128), jnp.float32),
        pltpu.VMEM((CW_ROWS, 128), jnp.float32),
        pltpu.VMEM((CW_ROWS, 128), jnp.float32),
        pltpu.VMEM((CW_ROWS, 128), jnp.float32),
        pltpu.VMEM_SHARED((8, 2, NR, 128), jnp.float32),
    ]

    @functools.partial(pl.kernel, mesh=_MESH, out_type=out_type,
                       scratch_types=scratch, compiler_params=_SC_PARAMS)
    def k(src_h, dst_h, dist_h, ea1_h, o_gs, ew_h, c_h,
          s_v, d_v, w_v, a_v, ewb, ccb,
          loc_first, loc_down, loc_dist, loc_indeg, loc_outdeg, loc_ewsum,
          accA, accB, bufA, bufB, sh):
        c = lax.axis_index("c")
        s = lax.axis_index("s")
        wid = s * NC + c
        base = wid * EPT
        pltpu.sync_copy(src_h.at[pl.ds(base, EPT)], s_v)
        pltpu.sync_copy(dst_h.at[pl.ds(base, EPT)], d_v)
        pltpu.sync_copy(dist_h.at[pl.ds(base, EPT)], w_v)
        pltpu.sync_copy(ea1_h.at[pl.ds(base, EPT)], a_v)

        def init_body(j, _):
            _v2s(loc_first, j, jnp.full((LN,), INIT_FIRST, jnp.float32))
            _v2s(loc_down, j, jnp.zeros((LN,), jnp.float32))
            _v2s(loc_dist, j, jnp.ones((LN,), jnp.float32))
            _v2s(loc_indeg, j, jnp.zeros((LN,), jnp.float32))
            _v2s(loc_outdeg, j, jnp.zeros((LN,), jnp.float32))
            _v2s(loc_ewsum, j, jnp.zeros((LN,), jnp.float32))
            return 0
        lax.fori_loop(0, NR * 8, init_body, 0)

        def edge_body(v, _):
            off = v * LN
            sl = pl.ds(off, LN)
            s16 = s_v[sl]
            d16 = d_v[sl]
            wraw = w_v[sl]
            a16 = a_v[sl]
            eid = base + off + _iota16()
            eid_u = eid.astype(jnp.uint32)
            valid = eid < E
            distc = jnp.maximum(wraw, jnp.float32(1e-6))
            ew16 = jnp.where(valid, 1.0 / distc, 0.0)
            ewb[sl] = ew16
            ccb[sl] = ew16 * a16
            kd = (d16.astype(jnp.uint32) << 18) | eid_u
            kds, dist_s = plsc.sort_key_val(kd, distc)
            dkey = lax.shift_right_logical(kds, jnp.uint32(18)).astype(jnp.int32)
            eid_s = (kds & jnp.uint32(0x3FFFF)).astype(jnp.int32)
            fst, lst = _run_masks(dkey)
            w1 = jnp.where(eid_s < E, jnp.float32(1.0), jnp.float32(0.0))
            ews = jnp.where(eid_s < E, 1.0 / dist_s, 0.0)
            _seg_add_into2(loc_indeg, dkey, w1, fst, lst)
            _seg_add_into2(loc_ewsum, dkey, ews, fst, lst)
            ks = (s16.astype(jnp.uint32) << 18) | eid_u
            kss, dst_s2 = plsc.sort_key_val(ks, d16.astype(jnp.float32))
            _, dist_s2 = plsc.sort_key_val(ks, distc)
            skey = lax.shift_right_logical(kss, jnp.uint32(18)).astype(jnp.int32)
            eid2 = (kss & jnp.uint32(0x3FFFF)).astype(jnp.int32)
            fst2, lst2 = _run_masks(skey)
            w2 = jnp.where(eid2 < E, jnp.float32(1.0), jnp.float32(0.0))
            _seg_add_into2(loc_outdeg, skey, w2, fst2, lst2)
            eid2f = eid2.astype(jnp.float32)
            r2i, c2i = _rc(skey)
            curF = plsc.load_gather(loc_first, [r2i, c2i], mask=fst2)
            win = fst2 & (eid2f < curF)
            plsc.store_scatter(loc_first, [r2i, c2i], eid2f, mask=win)
            plsc.store_scatter(loc_down, [r2i, c2i], dst_s2, mask=win)
            plsc.store_scatter(loc_dist, [r2i, c2i], dist_s2, mask=win)
            return 0
        lax.fori_loop(0, VE, edge_body, 0)

        pltpu.sync_copy(ewb, ew_h.at[pl.ds(base, EPT)])
        pltpu.sync_copy(ccb, c_h.at[pl.ds(base, EPT)])

        cstripe = pl.ds((s & 7) * CW_ROWS, CW_ROWS)

        def _minsel_round(locB, outA, outB, write_first):
            def rinit():
                @pl.when(s < NCONS)
                def _():
                    def zb(j, _):
                        _v2s(accA, j, jnp.full((LN,), INIT_FIRST, jnp.float32))
                        _v2s(accB, j, jnp.zeros((LN,), jnp.float32))
                        return 0
                    lax.fori_loop(0, CW_ROWS * 8, zb, 0)

            def rstep(i):
                pltpu.sync_copy(sh.at[i, 0, cstripe], bufA)
                pltpu.sync_copy(sh.at[i, 1, cstripe], bufB)

                def red(j, _):
                    a = _v2(accA, j)
                    bnew = _v2(bufA, j)
                    sel = bnew < a
                    _v2s(accA, j, jnp.where(sel, bnew, a))
                    _v2s(accB, j, jnp.where(sel, _v2(bufB, j), _v2(accB, j)))
                    return 0
                lax.fori_loop(0, CW_ROWS * 8, red, 0)

            def wout():
                @pl.when(s < NCONS)
                def _():
                    if write_first:
                        pltpu.sync_copy(accA, o_gs.at[c, outA, cstripe])
                    pltpu.sync_copy(accB, o_gs.at[c, outB, cstripe])

            _combine_rounds(sh, s, [loc_first, locB], rinit, rstep, wout)
            plsc.subcore_barrier()

        def _add_round(locsA, locB, outA, outB):
            def rinit():
                @pl.when(s < NCONS)
                def _():
                    def zb(j, _):
                        _v2s(accA, j, jnp.zeros((LN,), jnp.float32))
                        _v2s(accB, j, jnp.zeros((LN,), jnp.float32))
                        return 0
                    lax.fori_loop(0, CW_ROWS * 8, zb, 0)

            def rstep(i):
                pltpu.sync_copy(sh.at[i, 0, cstripe], bufA)
                pltpu.sync_copy(sh.at[i, 1, cstripe], bufB)

                def red(j, _):
                    _v2s(accA, j, _v2(accA, j) + _v2(bufA, j))
                    _v2s(accB, j, _v2(accB, j) + _v2(bufB, j))
                    return 0
                lax.fori_loop(0, CW_ROWS * 8, red, 0)

            def wout():
                @pl.when(s < NCONS)
                def _():
                    pltpu.sync_copy(accA, o_gs.at[c, outA, cstripe])
                    pltpu.sync_copy(accB, o_gs.at[c, outB, cstripe])

            _combine_rounds(sh, s, [locsA, locB], rinit, rstep, wout)
            plsc.subcore_barrier()

        _minsel_round(loc_down, 0, 1, True)
        _minsel_round(loc_dist, 0, 2, False)
        _add_round(loc_indeg, loc_outdeg, 3, 4)
        _add_round(loc_ewsum, loc_ewsum, 5, 5)

    return k


@functools.cache
def _gather_x8_kernel():
    CH = 160
    out_type = jax.ShapeDtypeStruct((TIN, NP, F), jnp.float32)
    scratch = [
        pltpu.VMEM((CH,), jnp.int32),
        pltpu.VMEM((CH, F), jnp.float32),
        pltpu.SemaphoreType.DMA,
    ]

    @functools.partial(pl.kernel, mesh=_MESH, out_type=out_type,
                       scratch_types=scratch, compiler_params=_SC_PARAMS)
    def k(x2d_h, idx8_h, out_h, idx_v, rows_v, sem):
        c = lax.axis_index("c")
        s = lax.axis_index("s")
        wid = s * NC + c
        base = wid * (NP // NW)

        def t_body(t, _):
            def ch_body(kk, _):
                pos = base + kk * CH
                pltpu.sync_copy(idx8_h.at[pl.ds(t * NP + pos, CH)], idx_v)
                pltpu.async_copy(x2d_h.at[idx_v], rows_v, sem).wait()
                pltpu.sync_copy(rows_v, out_h.at[t, pl.ds(pos, CH)])
                return 0
            lax.fori_loop(0, (NP // NW) // CH, ch_body, 0)
            return 0
        lax.fori_loop(0, TIN, t_body, 0)

    return k


_SC_PARAMS_SCTILE = pltpu.CompilerParams(needs_layout_passes=False,
                                         use_tc_tiling_on_sc=False)


@functools.cache
def _segsum_kernel(TR):
    out_type = jax.ShapeDtypeStruct((NC, NP, H), jnp.float32)
    scratch = [
        pltpu.VMEM((128,), jnp.int32),
        pltpu.VMEM((128,), jnp.int32),
        pltpu.VMEM((128,), jnp.float32),
        pltpu.VMEM((128, H), jnp.float32),
        pltpu.SemaphoreType.DMA,
        pltpu.VMEM_SHARED((NP, H), jnp.float32),
    ]

    @functools.partial(pl.kernel, mesh=_MESH, out_type=out_type,
                       scratch_types=scratch,
                       compiler_params=_SC_PARAMS_SCTILE)
    def k(h_h, src2_h, dst2_h, ew2_h, part_h, si_v, di_v, ew_v, rows_v, sem,
          acc_sh):
        c = lax.axis_index("c")
        s = lax.axis_index("s")
        wid = s * NC + c

        def z_body(r, _):
            for j in range(H // LN):
                rows_v[r, pl.ds(j * LN, LN)] = jnp.zeros((LN,), jnp.float32)
            return 0
        lax.fori_loop(0, 128, z_body, 0)

        def zc_body(kk, _):
            pltpu.sync_copy(rows_v,
                            acc_sh.at[pl.ds(s * (NP // NS) + kk * 128, 128)])
            return 0
        lax.fori_loop(0, (NP // NS) // 128, zc_body, 0)
        plsc.subcore_barrier()

        nrows = jnp.where(wid < RR - (RR // NW) * NW, RR // NW + 1, RR // NW)

        def row_body(i, _):
            row = wid + i * NW
            pltpu.sync_copy(src2_h.at[pl.ds(row * 128, 128)], si_v)
            pltpu.sync_copy(dst2_h.at[pl.ds(row * 128, 128)], di_v)
            pltpu.sync_copy(ew2_h.at[pl.ds(row * 128, 128)], ew_v)
            pltpu.async_copy(h_h.at[si_v], rows_v, sem).wait()

            def sc_body(g, _):
                r0 = g * LN
                ev16 = ew_v[pl.ds(r0, LN)]
                for kk in range(LN):
                    ev = ev16[kk]
                    for j in range(H // LN):
                        sl = pl.ds(j * LN, LN)
                        rows_v[r0 + kk, sl] = rows_v[r0 + kk, sl] * ev
                return 0
            lax.fori_loop(0, 128 // LN, sc_body, 0)
            pltpu.sync_copy(rows_v, acc_sh.at[di_v], add=True)
            return 0
        lax.fori_loop(0, nrows, row_body, 0)
        plsc.subcore_barrier()

        def out_body(kk, _):
            sl = pl.ds(s * (NP // NS) + kk * 128, 128)
            pltpu.sync_copy(acc_sh.at[sl], rows_v)
            pltpu.sync_copy(rows_v, part_h.at[c, sl])
            return 0
        lax.fori_loop(0, (NP // NS) // 128, out_body, 0)

    return k


def _stage_edges(src_h, dst_h, w_h, s_v, d_v, w_v, base):
    pltpu.sync_copy(src_h.at[pl.ds(base, EPT)], s_v)
    pltpu.sync_copy(dst_h.at[pl.ds(base, EPT)], d_v)
    pltpu.sync_copy(w_h.at[pl.ds(base, EPT)], w_v)


def _physics_partials(nx_v, s_v, d_v, w_v, cc_v, locd, locs):
    _zero2(locd, NR)
    _zero2(locs, NR)

    def e_body(v, _):
        sl = pl.ds(v * LN, LN)
        s16 = s_v[sl]
        d16 = d_v[sl]
        rs, cs = _rc(s16)
        rd, cd = _rc(d16)
        f16 = ((plsc.load_gather(nx_v, [rs, cs])
                - plsc.load_gather(nx_v, [rd, cd]))
               * w_v[sl] + cc_v[sl])
        dk, vs = plsc.sort_key_val(d16, f16)
        fst, lst = _run_masks(dk)
        _seg_add_into2(locd, dk, vs, fst, lst)
        sk, vs2 = plsc.sort_key_val(s16, f16)
        fst2, lst2 = _run_masks(sk)
        _seg_add_into2(locs, sk, vs2, fst2, lst2)
        return 0
    lax.fori_loop(0, VE, e_body, 0)


def _combine_one(loc2, sh, s, accs, bufs, write_out):
    @pl.when(s < NCONS)
    def _():
        def zb(j, _):
            _v2s(accs, j, jnp.zeros((LN,), jnp.float32))
            return 0
        lax.fori_loop(0, CW_ROWS * 8, zb, 0)
    cstripe = pl.ds((s & 7) * CW_ROWS, CW_ROWS)
    for g in range(2):
        @pl.when(lax.shift_right_logical(s, 3) == g)
        def _():
            pltpu.sync_copy(loc2, sh.at[s & 7])
        plsc.subcore_barrier()

        @pl.when(s < NCONS)
        def _():
            def cb(i, _):
                pltpu.sync_copy(sh.at[i, cstripe], bufs)

                def red(j, _):
                    _v2s(accs, j, _v2(accs, j) + _v2(bufs, j))
                    return 0
                lax.fori_loop(0, CW_ROWS * 8, red, 0)
                return 0
            lax.fori_loop(0, 8, cb, 0)
        plsc.subcore_barrier()

    @pl.when(s < NCONS)
    def _():
        write_out(cstripe)
    plsc.subcore_barrier()


def _publish_full(cb, shv, nx_v, s, out_h, c, estripe):
    @pl.when(s < 10)
    def _():
        pltpu.sync_copy(cb, shv.at[estripe])

        @pl.when(c == 0)
        def _():
            pltpu.sync_copy(cb, out_h.at[estripe])
    plsc.subcore_barrier()
    pltpu.sync_copy(shv, nx_v)


def _phys_scratch():
    return [
        pltpu.VMEM((NR, 128), jnp.float32),
        pltpu.VMEM((EW_ROWS, 128), jnp.float32),
        pltpu.VMEM((EW_ROWS, 128), jnp.float32),
        pltpu.VMEM((EW_ROWS, 128), jnp.float32),
        pltpu.VMEM((EPT,), jnp.int32),
        pltpu.VMEM((EPT,), jnp.int32),
        pltpu.VMEM((EPT,), jnp.float32),
        pltpu.VMEM((EPT,), jnp.float32),
        pltpu.VMEM((NR, 128), jnp.float32),
        pltpu.VMEM((NR, 128), jnp.float32),
        pltpu.VMEM((CW_ROWS, 128), jnp.float32),
        pltpu.VMEM((CW_ROWS, 128), jnp.float32),
        pltpu.VMEM_SHARED((NR, 128), jnp.float32),
        pltpu.VMEM_SHARED((8, NR, 128), jnp.float32),
    ]


@functools.cache
def _dec_l0_kernel():
    out_type = (
        jax.ShapeDtypeStruct((NR, 128), jnp.float32),
        jax.ShapeDtypeStruct((NC, NR, 128), jnp.float32),
    )
    scratch = [
        pltpu.VMEM((NR, 128), jnp.float32),
        pltpu.VMEM((EW_ROWS, 128), jnp.float32),
        pltpu.VMEM((EW_ROWS, 128), jnp.float32),
        pltpu.VMEM((EW_ROWS, 128), jnp.float32),
        pltpu.VMEM((EW_ROWS, 128), jnp.float32),
        pltpu.VMEM((EW_ROWS, 128), jnp.float32),
        pltpu.VMEM((EW_ROWS, 128), jnp.float32),
        pltpu.VMEM((EW_ROWS, 128), jnp.float32),
        pltpu.VMEM((16,), jnp.float32),
        pltpu.VMEM((EPT,), jnp.int32),
        pltpu.VMEM((EPT,), jnp.int32),
        pltpu.VMEM((EPT,), jnp.float32),
        pltpu.VMEM((NR, 128), jnp.float32),
        pltpu.VMEM((CW_ROWS, 128), jnp.float32),
        pltpu.VMEM((CW_ROWS, 128), jnp.float32),
        pltpu.VMEM_SHARED((NR, 128), jnp.float32),
        pltpu.VMEM_SHARED((8, NR, 128), jnp.float32),
    ]

    @functools.partial(pl.kernel, mesh=_MESH, out_type=out_type,
                       scratch_types=scratch, compiler_params=_SC_PARAMS)
    def k(vb_h, parts_h, degi_h, dt_h, ew_h, src_h, dst_h, cur_h, sp_h,
          cur_v, cb, vb, dg, p00, p01, p10, p11, dtv,
          s_v, d_v, w_v, loc, accs, bufs, shv, shp):
        c = lax.axis_index("c")
        s = lax.axis_index("s")
        wid = s * NC + c
        estripe = pl.ds(s * EW_ROWS, EW_ROWS)

        @pl.when(s < 10)
        def _():
            pltpu.sync_copy(vb_h.at[estripe], vb)
            pltpu.sync_copy(degi_h.at[estripe], dg)
            pltpu.sync_copy(parts_h.at[0, 0, estripe], p00)
            pltpu.sync_copy(parts_h.at[0, 1, estripe], p01)
            pltpu.sync_copy(parts_h.at[1, 0, estripe], p10)
            pltpu.sync_copy(parts_h.at[1, 1, estripe], p11)
            pltpu.sync_copy(dt_h, dtv)

            def fin_body(j, _):
                du = (_v2(p00, j) + _v2(p10, j) - _v2(p01, j) - _v2(p11, j)) \
                    * _v2(dg, j)
                _v2s(cb, j, _v2(vb, j) + dtv[...] * du)
                return 0
            lax.fori_loop(0, EW_ROWS * 8, fin_body, 0)

        _publish_full(cb, shv, cur_v, s, cur_h, c, estripe)

        _stage_edges(src_h, dst_h, ew_h, s_v, d_v, w_v, wid * EPT)
        _zero2(loc, NR)

        def e_body(v, _):
            sl = pl.ds(v * LN, LN)
            s16 = s_v[sl]
            d16 = d_v[sl]
            rs, cs2 = _rc(s16)
            val = w_v[sl] * plsc.load_gather(cur_v, [rs, cs2])
            dk, vs = plsc.sort_key_val(d16, val)
            fst, lst = _run_masks(dk)
            _seg_add_into2(loc, dk, vs, fst, lst)
            return 0
        lax.fori_loop(0, VE, e_body, 0)

        def wout(cstripe):
            pltpu.sync_copy(accs, sp_h.at[c, cstripe])
        _combine_one(loc, shp, s, accs, bufs, wout)

    return k


@functools.cache
def _phys_a_kernel():
    out_type = (
        jax.ShapeDtypeStruct((NR, 128), jnp.float32),
        jax.ShapeDtypeStruct((NC, 2, NR, 128), jnp.float32),
    )
    scratch = _phys_scratch() + [
        pltpu.VMEM((NR, 128), jnp.float32),
        pltpu.VMEM((EW_ROWS, 128), jnp.int32),
        pltpu.VMEM((176,), jnp.float32),
    ]

    @functools.partial(pl.kernel, mesh=_MESH, out_type=out_type,
                       scratch_types=scratch, compiler_params=_SC_PARAMS)
    def k(y_h, down_h, ivx_h, bc_h, w_h, ew_h, cc_h, src_h, dst_h,
          nxt_h, parts_h,
          nx_v, cb, ax1, ax2, s_v, d_v, w_v, cc_v, locd, locs, accs, bufs,
          shv, shp, y_v, dn, wv):
        c = lax.axis_index("c")
        s = lax.axis_index("s")
        wid = s * NC + c
        estripe = pl.ds(s * EW_ROWS, EW_ROWS)
        pltpu.sync_copy(y_h, y_v)
        pltpu.sync_copy(w_h, wv)

        wregs = [wv[pl.ds(16 * i, 16)] for i in range(11)]

        def _w(i):
            return wregs[i // 16][i % 16]

        @pl.when(s < 10)
        def _():
            pltpu.sync_copy(down_h.at[estripe], dn)
            pltpu.sync_copy(ivx_h.at[estripe], ax1)
            pltpu.sync_copy(bc_h.at[estripe], ax2)

            def f_body(j, _):
                y16 = _v2(y_v, s * EW_ROWS * 8 + j)
                dn16 = _v2(dn, j)
                rdn, cdn = _rc(dn16)
                yd = plsc.load_gather(y_v, [rdn, cdn])
                diff = (y16 - yd) * _v2(ax1, j)
                acc = jnp.full((LN,), 0.0, jnp.float32) + _w(160)
                for jj in range(32):
                    hj = jnp.maximum(
                        _w(3 * jj) * y16 + _w(3 * jj + 1) * yd
                        + _w(3 * jj + 2) * diff + _w(96 + jj),
                        jnp.float32(0.0))
                    acc = acc + _w(128 + jj) * hj
                _v2s(cb, j, jnp.where(_v2(ax2, j) > 0.5,
                                      y16 + ALPHA * acc, y16))
                return 0
            lax.fori_loop(0, EW_ROWS * 8, f_body, 0)

        _publish_full(cb, shv, nx_v, s, nxt_h, c, estripe)

        base = wid * EPT
        _stage_edges(src_h, dst_h, ew_h, s_v, d_v, w_v, base)
        pltpu.sync_copy(cc_h.at[pl.ds(base, EPT)], cc_v)
        _physics_partials(nx_v, s_v, d_v, w_v, cc_v, locd, locs)

        def wout_d(cstripe):
            pltpu.sync_copy(accs, parts_h.at[c, 0, cstripe])
        _combine_one(locd, shp, s, accs, bufs, wout_d)

        def wout_s(cstripe):
            pltpu.sync_copy(accs, parts_h.at[c, 1, cstripe])
        _combine_one(locs, shp, s, accs, bufs, wout_s)

    return k


@functools.cache
def _phys_b_kernel():
    out_type = (
        jax.ShapeDtypeStruct((NR, 128), jnp.float32),
        jax.ShapeDtypeStruct((NC, 2, NR, 128), jnp.float32),
    )
    scratch = _phys_scratch() + [
        pltpu.VMEM((EW_ROWS, 128), jnp.float32),
        pltpu.VMEM((EW_ROWS, 128), jnp.float32),
        pltpu.VMEM((EW_ROWS, 128), jnp.float32),
        pltpu.VMEM((EW_ROWS, 128), jnp.float32),
        pltpu.VMEM((EW_ROWS, 128), jnp.float32),
        pltpu.VMEM((16,), jnp.float32),
    ]

    @functools.partial(pl.kernel, mesh=_MESH, out_type=out_type,
                       scratch_types=scratch, compiler_params=_SC_PARAMS)
    def k(an_h, parts_h, bc_h, degi_h, dt_h, ew_h, cc_h, src_h, dst_h,
          v1r_h, parts2_h,
          nx_v, cb, ax1, ax2, s_v, d_v, w_v, cc_v, locd, locs, accs, bufs,
          shv, shp, anb, q00, q01, q10, q11, dtv):
        c = lax.axis_index("c")
        s = lax.axis_index("s")
        wid = s * NC + c
        estripe = pl.ds(s * EW_ROWS, EW_ROWS)

        @pl.when(s < 10)
        def _():
            pltpu.sync_copy(an_h.at[estripe], anb)
            pltpu.sync_copy(bc_h.at[estripe], ax1)
            pltpu.sync_copy(degi_h.at[estripe], ax2)
            pltpu.sync_copy(parts_h.at[0, 0, estripe], q00)
            pltpu.sync_copy(parts_h.at[0, 1, estripe], q01)
            pltpu.sync_copy(parts_h.at[1, 0, estripe], q10)
            pltpu.sync_copy(parts_h.at[1, 1, estripe], q11)
            pltpu.sync_copy(dt_h, dtv)

            def f_body(j, _):
                du = (_v2(q00, j) + _v2(q10, j) - _v2(q01, j) - _v2(q11, j)) \
                    * _v2(ax2, j)
                v1 = _v2(anb, j) + dtv[...] * du
                _v2s(cb, j, jnp.where(_v2(ax1, j) > 0.5,
                                      (1.0 - RELAX) * v1
                                      + RELAX * _v2(anb, j), v1))
                return 0
            lax.fori_loop(0, EW_ROWS * 8, f_body, 0)

        _publish_full(cb, shv, nx_v, s, v1r_h, c, estripe)

        base = wid * EPT
        _stage_edges(src_h, dst_h, ew_h, s_v, d_v, w_v, base)
        pltpu.sync_copy(cc_h.at[pl.ds(base, EPT)], cc_v)
        _physics_partials(nx_v, s_v, d_v, w_v, cc_v, locd, locs)

        def wout_d(cstripe):
            pltpu.sync_copy(accs, parts2_h.at[c, 0, cstripe])
        _combine_one(locd, shp, s, accs, bufs, wout_d)

        def wout_s(cstripe):
            pltpu.sync_copy(accs, parts2_h.at[c, 1, cstripe])
        _combine_one(locs, shp, s, accs, bufs, wout_s)

    return k


@functools.cache
def _tc_finalize():
    def body(gs_ref, down_ref, dx8_ref, ewni_ref, bcol_ref, ivx_ref,
             degi_ref):
        f0 = gs_ref[0, 0, :]
        f1 = gs_ref[1, 0, :]
        sel = f0 <= f1
        down = jnp.where(sel, gs_ref[0, 1, :], gs_ref[1, 1, :])
        distf = jnp.where(sel, gs_ref[0, 2, :], gs_ref[1, 2, :])
        indeg = gs_ref[0, 3, :] + gs_ref[1, 3, :]
        outdeg = gs_ref[0, 4, :] + gs_ref[1, 4, :]
        ewsum = gs_ref[0, 5, :] + gs_ref[1, 5, :]
        bmask = (indeg == 0.0) & (outdeg > 0.0)
        down_i = down.astype(jnp.int32)
        down_ref[...] = down_i
        dx8_ref[...] = (down_i[None, :] * TIN
                        + lax.broadcasted_iota(jnp.int32, (TIN, NP), 0))
        ewni_ref[...] = 1.0 / jnp.maximum(ewsum, 1e-6)
        bcol_ref[...] = jnp.where(bmask, 1.0, 0.0)
        ivx_ref[...] = 1.0 / jnp.maximum(distf, 1e-6)
        degi_ref[...] = 1.0 / (indeg + outdeg + 1.0)

    out_shape = (
        jax.ShapeDtypeStruct((NP,), jnp.int32),
        jax.ShapeDtypeStruct((TIN, NP), jnp.int32),
        jax.ShapeDtypeStruct((NP,), jnp.float32),
        jax.ShapeDtypeStruct((NP,), jnp.float32),
        jax.ShapeDtypeStruct((NP,), jnp.float32),
        jax.ShapeDtypeStruct((NP,), jnp.float32),
    )
    return pl.pallas_call(body, out_shape=out_shape)


_BN = 1280
_GRID = NP // _BN


def _row_spec(w):
    return pl.BlockSpec((_BN, w), lambda i: (i, 0))


def _part_spec(w):
    return pl.BlockSpec((NC, _BN, w), lambda i: (0, i, 0))


def _full_spec(shape):
    nd = len(shape)
    return pl.BlockSpec(shape, lambda i: (0,) * nd)


@functools.cache
def _tc_fuse_enc():
    def body(xt_ref, xd_ref, ivx_ref, bc_ref, w1_ref, b1_ref, w2_ref, b2_ref,
             o_ref):
        xt = xt_ref[...]
        xd = xd_ref[...]
        diff = (xt - xd) * ivx_ref[...]
        z = jnp.concatenate([xt, xd, diff], axis=1)
        h1 = jnp.maximum(jnp.dot(z, w1_ref[...],
                                 preferred_element_type=jnp.float32)
                         + b1_ref[...][None, :], 0.0)
        delta = jnp.dot(h1, w2_ref[...],
                        preferred_element_type=jnp.float32) + b2_ref[...][None, :]
        o_ref[...] = jnp.where(bc_ref[...] > 0.5, xt + ALPHA * delta, xt)

    return pl.pallas_call(
        body,
        out_shape=jax.ShapeDtypeStruct((NP, F), jnp.float32),
        grid=(_GRID,),
        in_specs=[_row_spec(F), _row_spec(F), _row_spec(1), _row_spec(1),
                  _full_spec((3 * F, FE)), _full_spec((FE,)),
                  _full_spec((FE, F)), _full_spec((F,))],
        out_specs=_row_spec(F),
    )


@functools.cache
def _tc_post128():
    def body(h_ref, pa_ref, pb_ref, ewni_ref, w_ref, b_ref, o_ref):
        ewni = ewni_ref[...]
        agg = jnp.concatenate([(pa_ref[0] + pa_ref[1]) * ewni,
                               (pb_ref[0] + pb_ref[1]) * ewni], axis=1)
        o_ref[...] = jnp.maximum(
            jnp.dot(h_ref[...] + agg, w_ref[...],
                    preferred_element_type=jnp.float32) + b_ref[...][None, :],
            0.0)

    return pl.pallas_call(
        body,
        out_shape=jax.ShapeDtypeStruct((NP, H), jnp.float32),
        grid=(_GRID,),
        in_specs=[_row_spec(F), _part_spec(H), _part_spec(H), _row_spec(1),
                  _full_spec((F, H)), _full_spec((H,))],
        out_specs=_row_spec(H),
    )


@functools.cache
def _tc_post64():
    def body(h_ref, p_ref, ewni_ref, w_ref, b_ref, o_ref):
        agg = (p_ref[0] + p_ref[1]) * ewni_ref[...]
        o_ref[...] = jnp.maximum(
            jnp.dot(h_ref[...] + agg, w_ref[...],
                    preferred_element_type=jnp.float32) + b_ref[...][None, :],
            0.0)

    return pl.pallas_call(
        body,
        out_shape=jax.ShapeDtypeStruct((NP, H), jnp.float32),
        grid=(_GRID,),
        in_specs=[_row_spec(H), _part_spec(H), _row_spec(1),
                  _full_spec((H, H)), _full_spec((H,))],
        out_specs=_row_spec(H),
    )


def _gru_math(xg, hs, wih_ref, whh_ref, bih_ref, bhh_ref):
    gi = jnp.dot(xg, wih_ref[...], preferred_element_type=jnp.float32) \
        + bih_ref[...][None, :]
    gh = jnp.dot(hs, whh_ref[...], preferred_element_type=jnp.float32) \
        + bhh_ref[...][None, :]
    r = jax.nn.sigmoid(gi[:, 0:H] + gh[:, 0:H])
    z = jax.nn.sigmoid(gi[:, H:2 * H] + gh[:, H:2 * H])
    n = jnp.tanh(gi[:, 2 * H:3 * H] + r * gh[:, 2 * H:3 * H])
    return (1.0 - z) * n + z * hs


@functools.cache
def _tc_post_gru():
    def body(h2_ref, p_ref, ewni_ref, w_ref, b_ref, hs_ref,
             wih_ref, whh_ref, bih_ref, bhh_ref, o_ref):
        agg = (p_ref[0] + p_ref[1]) * ewni_ref[...]
        xg = jnp.maximum(
            jnp.dot(h2_ref[...] + agg, w_ref[...],
                    preferred_element_type=jnp.float32) + b_ref[...][None, :],
            0.0)
        o_ref[...] = _gru_math(xg, hs_ref[...], wih_ref, whh_ref,
                               bih_ref, bhh_ref)

    return pl.pallas_call(
        body,
        out_shape=jax.ShapeDtypeStruct((NP, H), jnp.float32),
        grid=(_GRID,),
        in_specs=[_row_spec(H), _part_spec(H), _row_spec(1),
                  _full_spec((H, H)), _full_spec((H,)), _row_spec(H),
                  _full_spec((H, 3 * H)), _full_spec((H, 3 * H)),
                  _full_spec((3 * H,)), _full_spec((3 * H,))],
        out_specs=_row_spec(H),
    )


@functools.cache
def _tc_post_gru_head():
    def body(h2_ref, p_ref, ewni_ref, w_ref, b_ref, hs_ref,
             wih_ref, whh_ref, bih_ref, bhh_ref,
             wh1_ref, bh1_ref, wh2_ref, bh2_ref, wh3_ref, bh3_ref,
             h_ref, y_ref):
        agg = (p_ref[0] + p_ref[1]) * ewni_ref[...]
        xg = jnp.maximum(
            jnp.dot(h2_ref[...] + agg, w_ref[...],
                    preferred_element_type=jnp.float32) + b_ref[...][None, :],
            0.0)
        hnew = _gru_math(xg, hs_ref[...], wih_ref, whh_ref, bih_ref, bhh_ref)
        h_ref[...] = hnew
        t1 = jnp.maximum(jnp.dot(hnew, wh1_ref[...],
                                 preferred_element_type=jnp.float32)
                         + bh1_ref[...][None, :], 0.0)
        t2 = jnp.maximum(jnp.dot(t1, wh2_ref[...],
                                 preferred_element_type=jnp.float32)
                         + bh2_ref[...][None, :], 0.0)
        y_ref[...] = jnp.dot(t2, wh3_ref[...],
                             preferred_element_type=jnp.float32) \
            + bh3_ref[...][None, :]

    return pl.pallas_call(
        body,
        out_shape=(jax.ShapeDtypeStruct((NP, H), jnp.float32),
                   jax.ShapeDtypeStruct((NP, 1), jnp.float32)),
        grid=(_GRID,),
        in_specs=[_row_spec(H), _part_spec(H), _row_spec(1),
                  _full_spec((H, H)), _full_spec((H,)), _row_spec(H),
                  _full_spec((H, 3 * H)), _full_spec((H, 3 * H)),
                  _full_spec((3 * H,)), _full_spec((3 * H,)),
                  _full_spec((H, H)), _full_spec((H,)),
                  _full_spec((H, H)), _full_spec((H,)),
                  _full_spec((H, 1)), _full_spec((1,))],
        out_specs=(_row_spec(H), _row_spec(1)),
    )


@functools.cache
def _tc_dec_l0_post():
    def body(cur_ref, sp_ref, ewni_ref, ws_ref, b_ref, o_ref):
        agg = (sp_ref[0] + sp_ref[1]) * ewni_ref[...]
        v = cur_ref[...] + agg
        o_ref[...] = jnp.maximum(v * ws_ref[...][None, :] + b_ref[...][None, :],
                                 0.0)

    return pl.pallas_call(
        body,
        out_shape=jax.ShapeDtypeStruct((NP, H), jnp.float32),
        grid=(_GRID,),
        in_specs=[_row_spec(1), _part_spec(1), _row_spec(1),
                  _full_spec((H,)), _full_spec((H,))],
        out_specs=_row_spec(H),
    )


def kernel(x, edge_index, edge_attr, W_g0, b_g0, W_g1, b_g1, W_g2, b_g2, W_ih, W_hh, b_ih, b_hh, W_h1, b_h1, W_h2, b_h2, W_h3, b_h3, W_fe1, b_fe1, W_fe2, b_fe2, W_fd1, b_fd1, W_fd2, b_fd2, dt):
    f32 = jnp.float32
    srcp = jnp.pad(edge_index[0], (0, EP - E))
    dstp = jnp.pad(edge_index[1], (0, EP - E))
    distp = jnp.pad(edge_attr[:, 0], (0, EP - E), constant_values=1.0)
    ea1p = jnp.pad(edge_attr[:, 1], (0, EP - E))
    srcA1 = srcp * 2
    srcB1 = srcp * 2 + 1
    x_p = jnp.pad(x, ((0, NP - N), (0, 0), (0, 0)))
    x2d = x.reshape(N * TIN, F)
    dt16 = jnp.full((16,), dt, f32)
    wfd = jnp.concatenate([
        W_fd1.reshape(-1), b_fd1, W_fd2.reshape(-1), b_fd2,
        jnp.zeros((176 - 96 - 32 - 32 - 1,), f32)])
    wsum0 = jnp.sum(W_g0, axis=1)
    Wg0t = W_g0.T
    Wg1t = W_g1.T
    Wg2t = W_g2.T
    Wiht = W_ih.T
    Whht = W_hh.T
    Wh1t = W_h1.T
    Wh2t = W_h2.T
    Wh3t = W_h3.T
    Wfe1t = W_fe1.T
    Wfe2t = W_fe2.T

    o_gs, ew_e, c_e = _graph_static_kernel()(srcp, dstp, distp, ea1p)
    o_gs = o_gs.reshape(NC, 6, NP)
    down, dx8, ewni, bcolf, ivxf, degi = _tc_finalize()(o_gs)
    ewni_c = ewni.reshape(NP, 1)
    bcol_c = bcolf.reshape(NP, 1)
    ivx_c = ivxf.reshape(NP, 1)
    bcol_r = bcolf.reshape(NR, 128)
    ivx_r = ivxf.reshape(NR, 128)
    degi_r = degi.reshape(NR, 128)
    down_r = down.reshape(NR, 128)

    xg8 = _gather_x8_kernel()(x2d, dx8.reshape(-1))

    seg_h = _segsum_kernel(NP)
    seg_x = _segsum_kernel(2 * NP)
    post128 = _tc_post128()
    post64 = _tc_post64()
    fuse_enc = _tc_fuse_enc()
    post_gru = _tc_post_gru()
    post_gru_head = _tc_post_gru_head()
    dec_l0 = _dec_l0_kernel()
    dec_l0_post = _tc_dec_l0_post()
    phys_a = _phys_a_kernel()
    phys_b = _phys_b_kernel()

    h = jnp.zeros((NP, H), f32)
    xf = None
    for t in range(TIN):
        xt = x_p[:, t, :]
        xf = fuse_enc(xt, xg8[t], ivx_c, bcol_c, Wfe1t, b_fe1, Wfe2t, b_fe2)
        xf2 = xf.reshape(2 * NP, H)
        pa = seg_x(xf2, srcA1, dstp, ew_e)
        pb = seg_x(xf2, srcB1, dstp, ew_e)
        h1 = post128(xf, pa, pb, ewni_c, Wg0t, b_g0)
        p1 = seg_h(h1, srcp, dstp, ew_e)
        h2 = post64(h1, p1, ewni_c, Wg1t, b_g1)
        p2 = seg_h(h2, srcp, dstp, ew_e)
        h = post_gru(h2, p2, ewni_c, Wg2t, b_g2, h, Wiht, Whht, b_ih, b_hh)

    vbase = xf[:, 0].reshape(NR, 128)
    parts = jnp.zeros((NC, 2, NR, 128), f32)
    preds = []
    for t in range(TOUT):
        cur, sp = dec_l0(vbase, parts, degi_r, dt16, ew_e, srcp, dstp)
        h1 = dec_l0_post(cur.reshape(NP, 1), sp.reshape(NC, NP, 1), ewni_c,
                         wsum0, b_g0)
        p1 = seg_h(h1, srcp, dstp, ew_e)
        h2 = post64(h1, p1, ewni_c, Wg1t, b_g1)
        p2 = seg_h(h2, srcp, dstp, ew_e)
        h, y = post_gru_head(h2, p2, ewni_c, Wg2t, b_g2, h,
                             Wiht, Whht, b_ih, b_hh,
                             Wh1t, b_h1, Wh2t, b_h2, Wh3t, b_h3)
        preds.append(y[:N])
        if t < TOUT - 1:
            nxt0, partsA = phys_a(y.reshape(NR, 128), down_r, ivx_r, bcol_r,
                                  wfd, ew_e, c_e, srcp, dstp)
            vbase, parts = phys_b(nxt0, partsA, bcol_r, degi_r, dt16,
                                  ew_e, c_e, srcp, dstp)
    return jnp.concatenate(preds, axis=1)


# speedup vs baseline: 5.4068x; 1.4575x over previous
import functools

import jax
import jax.numpy as jnp
from jax import lax
from jax.experimental import pallas as pl
from jax.experimental.pallas import tpu as pltpu
from jax.experimental.pallas import tpu_sc as plsc

N = 10000
E = 160000
F = 128
FE = 256
H = 64
TIN = 8
TOUT = 4
ALPHA = 0.5
RELAX = 0.7
BG = 2

NC = 2
NS = 16
LN = 16
NW = NC * NS

NP = 10240
NR = NP // 128
EW_ROWS = 8
CW_ROWS = 16
NCONS = NR // CW_ROWS
EP = 160256
EPT = EP // NW
VE = EPT // LN
RR = EP // 128
INIT_FIRST = float(1 << 22)

_MESH = plsc.VectorSubcoreMesh(core_axis_name="c", subcore_axis_name="s",
                               num_cores=NC, num_subcores=NS)
_SC_PARAMS = pltpu.CompilerParams(needs_layout_passes=False)


def _iota16():
    return lax.iota(jnp.int32, LN)


def _rc(n):
    return lax.shift_right_logical(n, 7), n & 127


def _run_masks(keys_sorted):
    i = _iota16()
    prev = jnp.take_along_axis(keys_sorted, jnp.maximum(i - 1, 0), axis=0)
    nxt = jnp.take_along_axis(keys_sorted, jnp.minimum(i + 1, LN - 1), axis=0)
    is_first = (i == 0) | (keys_sorted != prev)
    is_last = (i == LN - 1) | (keys_sorted != nxt)
    return is_first, is_last


def _seg_sums(vals, is_first):
    i = _iota16()
    cs = plsc.cumsum(vals)
    first_idx = plsc.cummax(jnp.where(is_first, i, 0))
    csx = jnp.take_along_axis(cs, jnp.maximum(first_idx - 1, 0), axis=0)
    excl = jnp.where(first_idx == 0, jnp.float32(0.0), csx)
    return cs - excl


def _seg_add_into2(loc2, keys_sorted, vals_sorted, is_first, is_last):
    seg = _seg_sums(vals_sorted, is_first)
    r, cc = _rc(keys_sorted)
    cur = plsc.load_gather(loc2, [r, cc], mask=is_last)
    plsc.store_scatter(loc2, [r, cc], cur + seg, mask=is_last)


def _zero2(ref2, nrows):
    def zb(j, _):
        ref2[lax.shift_right_logical(j, 3),
             pl.ds((j & 7) * LN, LN)] = jnp.zeros((LN,), jnp.float32)
        return 0
    lax.fori_loop(0, nrows * 8, zb, 0)


def _v2(ref2, j):
    return ref2[lax.shift_right_logical(j, 3), pl.ds((j & 7) * LN, LN)]


def _v2s(ref2, j, val):
    ref2[lax.shift_right_logical(j, 3), pl.ds((j & 7) * LN, LN)] = val


def _combine_rounds(sh, s, pubs, reduce_init, reduce_step, write_out):
    na = len(pubs)
    reduce_init()
    for g in range(2):
        @pl.when(lax.shift_right_logical(s, 3) == g)
        def _():
            for a in range(na):
                pltpu.sync_copy(pubs[a], sh.at[s & 7, a])
        plsc.subcore_barrier()

        @pl.when(s < NCONS)
        def _():
            def cb(i, _):
                reduce_step(i)
                return 0
            lax.fori_loop(0, 8, cb, 0)
        plsc.subcore_barrier()
    write_out()


@functools.cache
def _graph_static_kernel():
    out_type = (
        jax.ShapeDtypeStruct((NC, 6, NR, 128), jnp.float32),
        jax.ShapeDtypeStruct((EP,), jnp.float32),
        jax.ShapeDtypeStruct((EP,), jnp.float32),
    )
    scratch = [
        pltpu.VMEM((EPT,), jnp.int32),
        pltpu.VMEM((EPT,), jnp.int32),
        pltpu.VMEM((EPT,), jnp.float32),
        pltpu.VMEM((EPT,), jnp.float32),
        pltpu.VMEM((EPT,), jnp.float32),
        pltpu.VMEM((EPT,), jnp.float32),
        pltpu.VMEM((NR, 128), jnp.float32),
        pltpu.VMEM((NR, 128), jnp.float32),
        pltpu.VMEM((NR, 128), jnp.float32),
        pltpu.VMEM((NR, 128), jnp.float32),
        pltpu.VMEM((NR, 128), jnp.float32),
        pltpu.VMEM((NR, 128), jnp.float32),
        pltpu.VMEM((CW_ROWS, 128), jnp.float32),
        pltpu.VMEM((CW_ROWS, 128), jnp.float32),
        pltpu.VMEM((CW_ROWS, 128), jnp.float32),
        pltpu.VMEM((CW_ROWS, 128), jnp.float32),
        pltpu.VMEM_SHARED((8, 2, NR, 128), jnp.float32),
    ]

    @functools.partial(pl.kernel, mesh=_MESH, out_type=out_type,
                       scratch_types=scratch, compiler_params=_SC_PARAMS)
    def k(src_h, dst_h, dist_h, ea1_h, o_gs, ew_h, c_h,
          s_v, d_v, w_v, a_v, ewb, ccb,
          loc_first, loc_down, loc_dist, loc_indeg, loc_outdeg, loc_ewsum,
          accA, accB, bufA, bufB, sh):
        c = lax.axis_index("c")
        s = lax.axis_index("s")
        wid = s * NC + c
        base = wid * EPT
        pltpu.sync_copy(src_h.at[pl.ds(base, EPT)], s_v)
        pltpu.sync_copy(dst_h.at[pl.ds(base, EPT)], d_v)
        pltpu.sync_copy(dist_h.at[pl.ds(base, EPT)], w_v)
        pltpu.sync_copy(ea1_h.at[pl.ds(base, EPT)], a_v)

        def init_body(j, _):
            _v2s(loc_first, j, jnp.full((LN,), INIT_FIRST, jnp.float32))
            _v2s(loc_down, j, jnp.zeros((LN,), jnp.float32))
            _v2s(loc_dist, j, jnp.ones((LN,), jnp.float32))
            _v2s(loc_indeg, j, jnp.zeros((LN,), jnp.float32))
            _v2s(loc_outdeg, j, jnp.zeros((LN,), jnp.float32))
            _v2s(loc_ewsum, j, jnp.zeros((LN,), jnp.float32))
            return 0
        lax.fori_loop(0, NR * 8, init_body, 0)

        def edge_body(v, _):
            off = v * LN
            sl = pl.ds(off, LN)
            s16 = s_v[sl]
            d16 = d_v[sl]
            wraw = w_v[sl]
            a16 = a_v[sl]
            eid = base + off + _iota16()
            eid_u = eid.astype(jnp.uint32)
            valid = eid < E
            distc = jnp.maximum(wraw, jnp.float32(1e-6))
            ew16 = jnp.where(valid, 1.0 / distc, 0.0)
            ewb[sl] = ew16
            ccb[sl] = ew16 * a16
            kd = (d16.astype(jnp.uint32) << 18) | eid_u
            kds, dist_s = plsc.sort_key_val(kd, distc)
            dkey = lax.shift_right_logical(kds, jnp.uint32(18)).astype(jnp.int32)
            eid_s = (kds & jnp.uint32(0x3FFFF)).astype(jnp.int32)
            fst, lst = _run_masks(dkey)
            w1 = jnp.where(eid_s < E, jnp.float32(1.0), jnp.float32(0.0))
            ews = jnp.where(eid_s < E, 1.0 / dist_s, 0.0)
            _seg_add_into2(loc_indeg, dkey, w1, fst, lst)
            _seg_add_into2(loc_ewsum, dkey, ews, fst, lst)
            ks = (s16.astype(jnp.uint32) << 18) | eid_u
            kss, dst_s2 = plsc.sort_key_val(ks, d16.astype(jnp.float32))
            _, dist_s2 = plsc.sort_key_val(ks, distc)
            skey = lax.shift_right_logical(kss, jnp.uint32(18)).astype(jnp.int32)
            eid2 = (kss & jnp.uint32(0x3FFFF)).astype(jnp.int32)
            fst2, lst2 = _run_masks(skey)
            w2 = jnp.where(eid2 < E, jnp.float32(1.0), jnp.float32(0.0))
            _seg_add_into2(loc_outdeg, skey, w2, fst2, lst2)
            eid2f = eid2.astype(jnp.float32)
            r2i, c2i = _rc(skey)
            curF = plsc.load_gather(loc_first, [r2i, c2i], mask=fst2)
            win = fst2 & (eid2f < curF)
            plsc.store_scatter(loc_first, [r2i, c2i], eid2f, mask=win)
            plsc.store_scatter(loc_down, [r2i, c2i], dst_s2, mask=win)
            plsc.store_scatter(loc_dist, [r2i, c2i], dist_s2, mask=win)
            return 0
        lax.fori_loop(0, VE, edge_body, 0)

        pltpu.sync_copy(ewb, ew_h.at[pl.ds(base, EPT)])
        pltpu.sync_copy(ccb, c_h.at[pl.ds(base, EPT)])

        cstripe = pl.ds((s & 7) * CW_ROWS, CW_ROWS)

        def _minsel_round(locB, outA, outB, write_first):
            def rinit():
                @pl.when(s < NCONS)
                def _():
                    def zb(j, _):
                        _v2s(accA, j, jnp.full((LN,), INIT_FIRST, jnp.float32))
                        _v2s(accB, j, jnp.zeros((LN,), jnp.float32))
                        return 0
                    lax.fori_loop(0, CW_ROWS * 8, zb, 0)

            def rstep(i):
                pltpu.sync_copy(sh.at[i, 0, cstripe], bufA)
                pltpu.sync_copy(sh.at[i, 1, cstripe], bufB)

                def red(j, _):
                    a = _v2(accA, j)
                    bnew = _v2(bufA, j)
                    sel = bnew < a
                    _v2s(accA, j, jnp.where(sel, bnew, a))
                    _v2s(accB, j, jnp.where(sel, _v2(bufB, j), _v2(accB, j)))
                    return 0
                lax.fori_loop(0, CW_ROWS * 8, red, 0)

            def wout():
                @pl.when(s < NCONS)
                def _():
                    if write_first:
                        pltpu.sync_copy(accA, o_gs.at[c, outA, cstripe])
                    pltpu.sync_copy(accB, o_gs.at[c, outB, cstripe])

            _combine_rounds(sh, s, [loc_first, locB], rinit, rstep, wout)
            plsc.subcore_barrier()

        def _add_round(locsA, locB, outA, outB):
            def rinit():
                @pl.when(s < NCONS)
                def _():
                    def zb(j, _):
                        _v2s(accA, j, jnp.zeros((LN,), jnp.float32))
                        _v2s(accB, j, jnp.zeros((LN,), jnp.float32))
                        return 0
                    lax.fori_loop(0, CW_ROWS * 8, zb, 0)

            def rstep(i):
                pltpu.sync_copy(sh.at[i, 0, cstripe], bufA)
                pltpu.sync_copy(sh.at[i, 1, cstripe], bufB)

                def red(j, _):
                    _v2s(accA, j, _v2(accA, j) + _v2(bufA, j))
                    _v2s(accB, j, _v2(accB, j) + _v2(bufB, j))
                    return 0
                lax.fori_loop(0, CW_ROWS * 8, red, 0)

            def wout():
                @pl.when(s < NCONS)
                def _():
                    pltpu.sync_copy(accA, o_gs.at[c, outA, cstripe])
                    pltpu.sync_copy(accB, o_gs.at[c, outB, cstripe])

            _combine_rounds(sh, s, [locsA, locB], rinit, rstep, wout)
            plsc.subcore_barrier()

        _minsel_round(loc_down, 0, 1, True)
        _minsel_round(loc_dist, 0, 2, False)
        _add_round(loc_indeg, loc_outdeg, 3, 4)
        _add_round(loc_ewsum, loc_ewsum, 5, 5)

    return k


@functools.cache
def _gather_x8_kernel():
    CH = 160
    out_type = jax.ShapeDtypeStruct((TIN, NP, F), jnp.float32)
    scratch = [
        pltpu.VMEM((CH,), jnp.int32),
        pltpu.VMEM((CH, F), jnp.float32),
        pltpu.SemaphoreType.DMA,
    ]

    @functools.partial(pl.kernel, mesh=_MESH, out_type=out_type,
                       scratch_types=scratch, compiler_params=_SC_PARAMS)
    def k(x2d_h, idx8_h, out_h, idx_v, rows_v, sem):
        c = lax.axis_index("c")
        s = lax.axis_index("s")
        wid = s * NC + c
        base = wid * (NP // NW)

        def t_body(t, _):
            def ch_body(kk, _):
                pos = base + kk * CH
                pltpu.sync_copy(idx8_h.at[pl.ds(t * NP + pos, CH)], idx_v)
                pltpu.async_copy(x2d_h.at[idx_v], rows_v, sem).wait()
                pltpu.sync_copy(rows_v, out_h.at[t, pl.ds(pos, CH)])
                return 0
            lax.fori_loop(0, (NP // NW) // CH, ch_body, 0)
            return 0
        lax.fori_loop(0, TIN, t_body, 0)

    return k


_SC_PARAMS_SCTILE = pltpu.CompilerParams(needs_layout_passes=False,
                                         use_tc_tiling_on_sc=False)


_SEG_MAXR = RR // NW + 1


@functools.cache
def _segsum_kernel(TR):
    out_type = jax.ShapeDtypeStruct((NC, NP, H), jnp.float32)
    scratch = [
        pltpu.VMEM((_SEG_MAXR * 128,), jnp.int32),
        pltpu.VMEM((128,), jnp.int32),
        pltpu.VMEM((_SEG_MAXR * 128,), jnp.float32),
        pltpu.VMEM((2, 128, H), jnp.float32),
        pltpu.SemaphoreType.DMA,
        pltpu.SemaphoreType.DMA,
        pltpu.VMEM_SHARED((NP, H), jnp.float32),
    ]

    @functools.partial(pl.kernel, mesh=_MESH, out_type=out_type,
                       scratch_types=scratch,
                       compiler_params=_SC_PARAMS_SCTILE)
    def k(h_h, src2_h, dst2_h, ew2_h, part_h, si_v, di_v, ew_v, rows_v,
          sem0, sem1, acc_sh):
        c = lax.axis_index("c")
        s = lax.axis_index("s")
        wid = s * NC + c
        nrows = jnp.where(wid < RR - (RR // NW) * NW, RR // NW + 1, RR // NW)
        start = wid * (RR // NW) + jnp.minimum(wid, RR - (RR // NW) * NW)

        pltpu.sync_copy(src2_h.at[pl.ds(start * 128, _SEG_MAXR * 128)], si_v)
        pltpu.sync_copy(ew2_h.at[pl.ds(start * 128, _SEG_MAXR * 128)], ew_v)

        def z_body(r, _):
            for j in range(H // LN):
                rows_v[0, r, pl.ds(j * LN, LN)] = jnp.zeros((LN,), jnp.float32)
            return 0
        lax.fori_loop(0, 128, z_body, 0)

        def zc_body(kk, _):
            pltpu.sync_copy(rows_v.at[0],
                            acc_sh.at[pl.ds(s * (NP // NS) + kk * 128, 128)])
            return 0
        lax.fori_loop(0, (NP // NS) // 128, zc_body, 0)
        plsc.subcore_barrier()

        def _gather(i, b):
            pltpu.make_async_copy(
                h_h.at[si_v.at[pl.ds(i * 128, 128)]],
                rows_v.at[b], sem0 if b == 0 else sem1).start()

        def _gwait(b):
            pltpu.make_async_copy(
                h_h.at[si_v.at[pl.ds(0, 128)]],
                rows_v.at[b], sem0 if b == 0 else sem1).wait()

        _gather(0, 0)

        def row_body(i, _):
            nxt = i + 1

            @pl.when((nxt < nrows) & ((nxt & 1) == 0))
            def _():
                _gather(nxt, 0)

            @pl.when((nxt < nrows) & ((nxt & 1) == 1))
            def _():
                _gather(nxt, 1)
            pltpu.sync_copy(dst2_h.at[pl.ds((start + i) * 128, 128)], di_v)

            @pl.when((i & 1) == 0)
            def _():
                _gwait(0)

            @pl.when((i & 1) == 1)
            def _():
                _gwait(1)
            b = i & 1

            def sc_body(g, _):
                r0 = g * LN
                ev16 = ew_v[pl.ds(i * 128 + r0, LN)]
                for kk in range(LN):
                    ev = ev16[kk]
                    for j in range(H // LN):
                        sl = pl.ds(j * LN, LN)
                        rows_v[b, r0 + kk, sl] = rows_v[b, r0 + kk, sl] * ev
                return 0
            lax.fori_loop(0, 128 // LN, sc_body, 0)
            pltpu.sync_copy(rows_v.at[b], acc_sh.at[di_v], add=True)
            return 0
        lax.fori_loop(0, nrows, row_body, 0)
        plsc.subcore_barrier()

        def out_body(kk, _):
            sl = pl.ds(s * (NP // NS) + kk * 128, 128)
            pltpu.sync_copy(acc_sh.at[sl], rows_v.at[0])
            pltpu.sync_copy(rows_v.at[0], part_h.at[c, sl])
            return 0
        lax.fori_loop(0, (NP // NS) // 128, out_body, 0)

    return k


def _stage_edges(src_h, dst_h, w_h, s_v, d_v, w_v, base):
    pltpu.sync_copy(src_h.at[pl.ds(base, EPT)], s_v)
    pltpu.sync_copy(dst_h.at[pl.ds(base, EPT)], d_v)
    pltpu.sync_copy(w_h.at[pl.ds(base, EPT)], w_v)


def _physics_partials(nx_v, s_v, d_v, w_v, cc_v, locd, locs):
    _zero2(locd, NR)
    _zero2(locs, NR)

    def e_body(v, _):
        sl = pl.ds(v * LN, LN)
        s16 = s_v[sl]
        d16 = d_v[sl]
        rs, cs = _rc(s16)
        rd, cd = _rc(d16)
        f16 = ((plsc.load_gather(nx_v, [rs, cs])
                - plsc.load_gather(nx_v, [rd, cd]))
               * w_v[sl] + cc_v[sl])
        dk, vs = plsc.sort_key_val(d16, f16)
        fst, lst = _run_masks(dk)
        _seg_add_into2(locd, dk, vs, fst, lst)
        sk, vs2 = plsc.sort_key_val(s16, f16)
        fst2, lst2 = _run_masks(sk)
        _seg_add_into2(locs, sk, vs2, fst2, lst2)
        return 0
    lax.fori_loop(0, VE, e_body, 0)


def _combine_one(loc2, sh, s, accs, bufs, write_out):
    @pl.when(s < NCONS)
    def _():
        def zb(j, _):
            _v2s(accs, j, jnp.zeros((LN,), jnp.float32))
            return 0
        lax.fori_loop(0, CW_ROWS * 8, zb, 0)
    cstripe = pl.ds((s & 7) * CW_ROWS, CW_ROWS)
    for g in range(2):
        @pl.when(lax.shift_right_logical(s, 3) == g)
        def _():
            pltpu.sync_copy(loc2, sh.at[s & 7])
        plsc.subcore_barrier()

        @pl.when(s < NCONS)
        def _():
            def cb(i, _):
                pltpu.sync_copy(sh.at[i, cstripe], bufs)

                def red(j, _):
                    _v2s(accs, j, _v2(accs, j) + _v2(bufs, j))
                    return 0
                lax.fori_loop(0, CW_ROWS * 8, red, 0)
                return 0
            lax.fori_loop(0, 8, cb, 0)
        plsc.subcore_barrier()

    @pl.when(s < NCONS)
    def _():
        write_out(cstripe)
    plsc.subcore_barrier()


def _publish_full(cb, shv, nx_v, s, out_h, c, estripe):
    @pl.when(s < 10)
    def _():
        pltpu.sync_copy(cb, shv.at[estripe])

        @pl.when(c == 0)
        def _():
            pltpu.sync_copy(cb, out_h.at[estripe])
    plsc.subcore_barrier()
    pltpu.sync_copy(shv, nx_v)


def _phys_scratch():
    return [
        pltpu.VMEM((NR, 128), jnp.float32),
        pltpu.VMEM((EW_ROWS, 128), jnp.float32),
        pltpu.VMEM((EW_ROWS, 128), jnp.float32),
        pltpu.VMEM((EW_ROWS, 128), jnp.float32),
        pltpu.VMEM((EPT,), jnp.int32),
        pltpu.VMEM((EPT,), jnp.int32),
        pltpu.VMEM((EPT,), jnp.float32),
        pltpu.VMEM((EPT,), jnp.float32),
        pltpu.VMEM((NR, 128), jnp.float32),
        pltpu.VMEM((NR, 128), jnp.float32),
        pltpu.VMEM((CW_ROWS, 128), jnp.float32),
        pltpu.VMEM((CW_ROWS, 128), jnp.float32),
        pltpu.VMEM_SHARED((NR, 128), jnp.float32),
        pltpu.VMEM_SHARED((8, NR, 128), jnp.float32),
    ]


@functools.cache
def _dec_l0_kernel():
    out_type = (
        jax.ShapeDtypeStruct((NR, 128), jnp.float32),
        jax.ShapeDtypeStruct((NC, NR, 128), jnp.float32),
    )
    scratch = [
        pltpu.VMEM((NR, 128), jnp.float32),
        pltpu.VMEM((EW_ROWS, 128), jnp.float32),
        pltpu.VMEM((EW_ROWS, 128), jnp.float32),
        pltpu.VMEM((EW_ROWS, 128), jnp.float32),
        pltpu.VMEM((EW_ROWS, 128), jnp.float32),
        pltpu.VMEM((EW_ROWS, 128), jnp.float32),
        pltpu.VMEM((EW_ROWS, 128), jnp.float32),
        pltpu.VMEM((EW_ROWS, 128), jnp.float32),
        pltpu.VMEM((16,), jnp.float32),
        pltpu.VMEM((EPT,), jnp.int32),
        pltpu.VMEM((EPT,), jnp.int32),
        pltpu.VMEM((EPT,), jnp.float32),
        pltpu.VMEM((NR, 128), jnp.float32),
        pltpu.VMEM((CW_ROWS, 128), jnp.float32),
        pltpu.VMEM((CW_ROWS, 128), jnp.float32),
        pltpu.VMEM_SHARED((NR, 128), jnp.float32),
        pltpu.VMEM_SHARED((8, NR, 128), jnp.float32),
    ]

    @functools.partial(pl.kernel, mesh=_MESH, out_type=out_type,
                       scratch_types=scratch, compiler_params=_SC_PARAMS)
    def k(vb_h, parts_h, degi_h, dt_h, ew_h, src_h, dst_h, cur_h, sp_h,
          cur_v, cb, vb, dg, p00, p01, p10, p11, dtv,
          s_v, d_v, w_v, loc, accs, bufs, shv, shp):
        c = lax.axis_index("c")
        s = lax.axis_index("s")
        wid = s * NC + c
        estripe = pl.ds(s * EW_ROWS, EW_ROWS)

        @pl.when(s < 10)
        def _():
            pltpu.sync_copy(vb_h.at[estripe], vb)
            pltpu.sync_copy(degi_h.at[estripe], dg)
            pltpu.sync_copy(parts_h.at[0, 0, estripe], p00)
            pltpu.sync_copy(parts_h.at[0, 1, estripe], p01)
            pltpu.sync_copy(parts_h.at[1, 0, estripe], p10)
            pltpu.sync_copy(parts_h.at[1, 1, estripe], p11)
            pltpu.sync_copy(dt_h, dtv)

            def fin_body(j, _):
                du = (_v2(p00, j) + _v2(p10, j) - _v2(p01, j) - _v2(p11, j)) \
                    * _v2(dg, j)
                _v2s(cb, j, _v2(vb, j) + dtv[...] * du)
                return 0
            lax.fori_loop(0, EW_ROWS * 8, fin_body, 0)

        _publish_full(cb, shv, cur_v, s, cur_h, c, estripe)

        _stage_edges(src_h, dst_h, ew_h, s_v, d_v, w_v, wid * EPT)
        _zero2(loc, NR)

        def e_body(v, _):
            sl = pl.ds(v * LN, LN)
            s16 = s_v[sl]
            d16 = d_v[sl]
            rs, cs2 = _rc(s16)
            val = w_v[sl] * plsc.load_gather(cur_v, [rs, cs2])
            dk, vs = plsc.sort_key_val(d16, val)
            fst, lst = _run_masks(dk)
            _seg_add_into2(loc, dk, vs, fst, lst)
            return 0
        lax.fori_loop(0, VE, e_body, 0)

        def wout(cstripe):
            pltpu.sync_copy(accs, sp_h.at[c, cstripe])
        _combine_one(loc, shp, s, accs, bufs, wout)

    return k


@functools.cache
def _phys_a_kernel():
    out_type = (
        jax.ShapeDtypeStruct((NR, 128), jnp.float32),
        jax.ShapeDtypeStruct((NC, 2, NR, 128), jnp.float32),
    )
    scratch = _phys_scratch() + [
        pltpu.VMEM((NR, 128), jnp.float32),
        pltpu.VMEM((EW_ROWS, 128), jnp.int32),
        pltpu.VMEM((176,), jnp.float32),
    ]

    @functools.partial(pl.kernel, mesh=_MESH, out_type=out_type,
                       scratch_types=scratch, compiler_params=_SC_PARAMS)
    def k(y_h, down_h, ivx_h, bc_h, w_h, ew_h, cc_h, src_h, dst_h,
          nxt_h, parts_h,
          nx_v, cb, ax1, ax2, s_v, d_v, w_v, cc_v, locd, locs, accs, bufs,
          shv, shp, y_v, dn, wv):
        c = lax.axis_index("c")
        s = lax.axis_index("s")
        wid = s * NC + c
        estripe = pl.ds(s * EW_ROWS, EW_ROWS)
        pltpu.sync_copy(y_h, y_v)
        pltpu.sync_copy(w_h, wv)

        wregs = [wv[pl.ds(16 * i, 16)] for i in range(11)]

        def _w(i):
            return wregs[i // 16][i % 16]

        @pl.when(s < 10)
        def _():
            pltpu.sync_copy(down_h.at[estripe], dn)
            pltpu.sync_copy(ivx_h.at[estripe], ax1)
            pltpu.sync_copy(bc_h.at[estripe], ax2)

            def f_body(j, _):
                y16 = _v2(y_v, s * EW_ROWS * 8 + j)
                dn16 = _v2(dn, j)
                rdn, cdn = _rc(dn16)
                yd = plsc.load_gather(y_v, [rdn, cdn])
                diff = (y16 - yd) * _v2(ax1, j)
                acc = jnp.full((LN,), 0.0, jnp.float32) + _w(160)
                for jj in range(32):
                    hj = jnp.maximum(
                        _w(3 * jj) * y16 + _w(3 * jj + 1) * yd
                        + _w(3 * jj + 2) * diff + _w(96 + jj),
                        jnp.float32(0.0))
                    acc = acc + _w(128 + jj) * hj
                _v2s(cb, j, jnp.where(_v2(ax2, j) > 0.5,
                                      y16 + ALPHA * acc, y16))
                return 0
            lax.fori_loop(0, EW_ROWS * 8, f_body, 0)

        _publish_full(cb, shv, nx_v, s, nxt_h, c, estripe)

        base = wid * EPT
        _stage_edges(src_h, dst_h, ew_h, s_v, d_v, w_v, base)
        pltpu.sync_copy(cc_h.at[pl.ds(base, EPT)], cc_v)
        _physics_partials(nx_v, s_v, d_v, w_v, cc_v, locd, locs)

        def wout_d(cstripe):
            pltpu.sync_copy(accs, parts_h.at[c, 0, cstripe])
        _combine_one(locd, shp, s, accs, bufs, wout_d)

        def wout_s(cstripe):
            pltpu.sync_copy(accs, parts_h.at[c, 1, cstripe])
        _combine_one(locs, shp, s, accs, bufs, wout_s)

    return k


@functools.cache
def _phys_b_kernel():
    out_type = (
        jax.ShapeDtypeStruct((NR, 128), jnp.float32),
        jax.ShapeDtypeStruct((NC, 2, NR, 128), jnp.float32),
    )
    scratch = _phys_scratch() + [
        pltpu.VMEM((EW_ROWS, 128), jnp.float32),
        pltpu.VMEM((EW_ROWS, 128), jnp.float32),
        pltpu.VMEM((EW_ROWS, 128), jnp.float32),
        pltpu.VMEM((EW_ROWS, 128), jnp.float32),
        pltpu.VMEM((EW_ROWS, 128), jnp.float32),
        pltpu.VMEM((16,), jnp.float32),
    ]

    @functools.partial(pl.kernel, mesh=_MESH, out_type=out_type,
                       scratch_types=scratch, compiler_params=_SC_PARAMS)
    def k(an_h, parts_h, bc_h, degi_h, dt_h, ew_h, cc_h, src_h, dst_h,
          v1r_h, parts2_h,
          nx_v, cb, ax1, ax2, s_v, d_v, w_v, cc_v, locd, locs, accs, bufs,
          shv, shp, anb, q00, q01, q10, q11, dtv):
        c = lax.axis_index("c")
        s = lax.axis_index("s")
        wid = s * NC + c
        estripe = pl.ds(s * EW_ROWS, EW_ROWS)

        @pl.when(s < 10)
        def _():
            pltpu.sync_copy(an_h.at[estripe], anb)
            pltpu.sync_copy(bc_h.at[estripe], ax1)
            pltpu.sync_copy(degi_h.at[estripe], ax2)
            pltpu.sync_copy(parts_h.at[0, 0, estripe], q00)
            pltpu.sync_copy(parts_h.at[0, 1, estripe], q01)
            pltpu.sync_copy(parts_h.at[1, 0, estripe], q10)
            pltpu.sync_copy(parts_h.at[1, 1, estripe], q11)
            pltpu.sync_copy(dt_h, dtv)

            def f_body(j, _):
                du = (_v2(q00, j) + _v2(q10, j) - _v2(q01, j) - _v2(q11, j)) \
                    * _v2(ax2, j)
                v1 = _v2(anb, j) + dtv[...] * du
                _v2s(cb, j, jnp.where(_v2(ax1, j) > 0.5,
                                      (1.0 - RELAX) * v1
                                      + RELAX * _v2(anb, j), v1))
                return 0
            lax.fori_loop(0, EW_ROWS * 8, f_body, 0)

        _publish_full(cb, shv, nx_v, s, v1r_h, c, estripe)

        base = wid * EPT
        _stage_edges(src_h, dst_h, ew_h, s_v, d_v, w_v, base)
        pltpu.sync_copy(cc_h.at[pl.ds(base, EPT)], cc_v)
        _physics_partials(nx_v, s_v, d_v, w_v, cc_v, locd, locs)

        def wout_d(cstripe):
            pltpu.sync_copy(accs, parts2_h.at[c, 0, cstripe])
        _combine_one(locd, shp, s, accs, bufs, wout_d)

        def wout_s(cstripe):
            pltpu.sync_copy(accs, parts2_h.at[c, 1, cstripe])
        _combine_one(locs, shp, s, accs, bufs, wout_s)

    return k


@functools.cache
def _tc_finalize():
    def body(gs_ref, down_ref, dx8_ref, ewni_ref, bcol_ref, ivx_ref,
             degi_ref):
        f0 = gs_ref[0, 0, :]
        f1 = gs_ref[1, 0, :]
        sel = f0 <= f1
        down = jnp.where(sel, gs_ref[0, 1, :], gs_ref[1, 1, :])
        distf = jnp.where(sel, gs_ref[0, 2, :], gs_ref[1, 2, :])
        indeg = gs_ref[0, 3, :] + gs_ref[1, 3, :]
        outdeg = gs_ref[0, 4, :] + gs_ref[1, 4, :]
        ewsum = gs_ref[0, 5, :] + gs_ref[1, 5, :]
        bmask = (indeg == 0.0) & (outdeg > 0.0)
        down_i = down.astype(jnp.int32)
        down_ref[...] = down_i
        dx8_ref[...] = (down_i[None, :] * TIN
                        + lax.broadcasted_iota(jnp.int32, (TIN, NP), 0))
        ewni_ref[...] = 1.0 / jnp.maximum(ewsum, 1e-6)
        bcol_ref[...] = jnp.where(bmask, 1.0, 0.0)
        ivx_ref[...] = 1.0 / jnp.maximum(distf, 1e-6)
        degi_ref[...] = 1.0 / (indeg + outdeg + 1.0)

    out_shape = (
        jax.ShapeDtypeStruct((NP,), jnp.int32),
        jax.ShapeDtypeStruct((TIN, NP), jnp.int32),
        jax.ShapeDtypeStruct((NP,), jnp.float32),
        jax.ShapeDtypeStruct((NP,), jnp.float32),
        jax.ShapeDtypeStruct((NP,), jnp.float32),
        jax.ShapeDtypeStruct((NP,), jnp.float32),
    )
    return pl.pallas_call(body, out_shape=out_shape)


_BN = 1280
_GRID = NP // _BN


def _row_spec(w):
    return pl.BlockSpec((_BN, w), lambda i: (i, 0))


def _part_spec(w):
    return pl.BlockSpec((NC, _BN, w), lambda i: (0, i, 0))


def _full_spec(shape):
    nd = len(shape)
    return pl.BlockSpec(shape, lambda i: (0,) * nd)


@functools.cache
def _tc_fuse_enc():
    def body(xt_ref, xd_ref, ivx_ref, bc_ref, w1_ref, b1_ref, w2_ref, b2_ref,
             o_ref):
        xt = xt_ref[...]
        xd = xd_ref[...]
        diff = (xt - xd) * ivx_ref[...]
        z = jnp.concatenate([xt, xd, diff], axis=1)
        h1 = jnp.maximum(jnp.dot(z, w1_ref[...],
                                 preferred_element_type=jnp.float32)
                         + b1_ref[...][None, :], 0.0)
        delta = jnp.dot(h1, w2_ref[...],
                        preferred_element_type=jnp.float32) + b2_ref[...][None, :]
        o_ref[...] = jnp.where(bc_ref[...] > 0.5, xt + ALPHA * delta, xt)

    return pl.pallas_call(
        body,
        out_shape=jax.ShapeDtypeStruct((NP, F), jnp.float32),
        grid=(_GRID,),
        in_specs=[_row_spec(F), _row_spec(F), _row_spec(1), _row_spec(1),
                  _full_spec((3 * F, FE)), _full_spec((FE,)),
                  _full_spec((FE, F)), _full_spec((F,))],
        out_specs=_row_spec(F),
    )


@functools.cache
def _tc_post128():
    def body(h_ref, pa_ref, pb_ref, ewni_ref, w_ref, b_ref, o_ref):
        ewni = ewni_ref[...]
        agg = jnp.concatenate([(pa_ref[0] + pa_ref[1]) * ewni,
                               (pb_ref[0] + pb_ref[1]) * ewni], axis=1)
        o_ref[...] = jnp.maximum(
            jnp.dot(h_ref[...] + agg, w_ref[...],
                    preferred_element_type=jnp.float32) + b_ref[...][None, :],
            0.0)

    return pl.pallas_call(
        body,
        out_shape=jax.ShapeDtypeStruct((NP, H), jnp.float32),
        grid=(_GRID,),
        in_specs=[_row_spec(F), _part_spec(H), _part_spec(H), _row_spec(1),
                  _full_spec((F, H)), _full_spec((H,))],
        out_specs=_row_spec(H),
    )


@functools.cache
def _tc_post64():
    def body(h_ref, p_ref, ewni_ref, w_ref, b_ref, o_ref):
        agg = (p_ref[0] + p_ref[1]) * ewni_ref[...]
        o_ref[...] = jnp.maximum(
            jnp.dot(h_ref[...] + agg, w_ref[...],
                    preferred_element_type=jnp.float32) + b_ref[...][None, :],
            0.0)

    return pl.pallas_call(
        body,
        out_shape=jax.ShapeDtypeStruct((NP, H), jnp.float32),
        grid=(_GRID,),
        in_specs=[_row_spec(H), _part_spec(H), _row_spec(1),
                  _full_spec((H, H)), _full_spec((H,))],
        out_specs=_row_spec(H),
    )


def _gru_math(xg, hs, wih_ref, whh_ref, bih_ref, bhh_ref):
    gi = jnp.dot(xg, wih_ref[...], preferred_element_type=jnp.float32) \
        + bih_ref[...][None, :]
    gh = jnp.dot(hs, whh_ref[...], preferred_element_type=jnp.float32) \
        + bhh_ref[...][None, :]
    r = jax.nn.sigmoid(gi[:, 0:H] + gh[:, 0:H])
    z = jax.nn.sigmoid(gi[:, H:2 * H] + gh[:, H:2 * H])
    n = jnp.tanh(gi[:, 2 * H:3 * H] + r * gh[:, 2 * H:3 * H])
    return (1.0 - z) * n + z * hs


@functools.cache
def _tc_post_gru():
    def body(h2_ref, p_ref, ewni_ref, w_ref, b_ref, hs_ref,
             wih_ref, whh_ref, bih_ref, bhh_ref, o_ref):
        agg = (p_ref[0] + p_ref[1]) * ewni_ref[...]
        xg = jnp.maximum(
            jnp.dot(h2_ref[...] + agg, w_ref[...],
                    preferred_element_type=jnp.float32) + b_ref[...][None, :],
            0.0)
        o_ref[...] = _gru_math(xg, hs_ref[...], wih_ref, whh_ref,
                               bih_ref, bhh_ref)

    return pl.pallas_call(
        body,
        out_shape=jax.ShapeDtypeStruct((NP, H), jnp.float32),
        grid=(_GRID,),
        in_specs=[_row_spec(H), _part_spec(H), _row_spec(1),
                  _full_spec((H, H)), _full_spec((H,)), _row_spec(H),
                  _full_spec((H, 3 * H)), _full_spec((H, 3 * H)),
                  _full_spec((3 * H,)), _full_spec((3 * H,))],
        out_specs=_row_spec(H),
    )


@functools.cache
def _tc_post_gru_head():
    def body(h2_ref, p_ref, ewni_ref, w_ref, b_ref, hs_ref,
             wih_ref, whh_ref, bih_ref, bhh_ref,
             wh1_ref, bh1_ref, wh2_ref, bh2_ref, wh3_ref, bh3_ref,
             h_ref, y_ref):
        agg = (p_ref[0] + p_ref[1]) * ewni_ref[...]
        xg = jnp.maximum(
            jnp.dot(h2_ref[...] + agg, w_ref[...],
                    preferred_element_type=jnp.float32) + b_ref[...][None, :],
            0.0)
        hnew = _gru_math(xg, hs_ref[...], wih_ref, whh_ref, bih_ref, bhh_ref)
        h_ref[...] = hnew
        t1 = jnp.maximum(jnp.dot(hnew, wh1_ref[...],
                                 preferred_element_type=jnp.float32)
                         + bh1_ref[...][None, :], 0.0)
        t2 = jnp.maximum(jnp.dot(t1, wh2_ref[...],
                                 preferred_element_type=jnp.float32)
                         + bh2_ref[...][None, :], 0.0)
        y_ref[...] = jnp.dot(t2, wh3_ref[...],
                             preferred_element_type=jnp.float32) \
            + bh3_ref[...][None, :]

    return pl.pallas_call(
        body,
        out_shape=(jax.ShapeDtypeStruct((NP, H), jnp.float32),
                   jax.ShapeDtypeStruct((NP, 1), jnp.float32)),
        grid=(_GRID,),
        in_specs=[_row_spec(H), _part_spec(H), _row_spec(1),
                  _full_spec((H, H)), _full_spec((H,)), _row_spec(H),
                  _full_spec((H, 3 * H)), _full_spec((H, 3 * H)),
                  _full_spec((3 * H,)), _full_spec((3 * H,)),
                  _full_spec((H, H)), _full_spec((H,)),
                  _full_spec((H, H)), _full_spec((H,)),
                  _full_spec((H, 1)), _full_spec((1,))],
        out_specs=(_row_spec(H), _row_spec(1)),
    )


@functools.cache
def _tc_dec_l0_post():
    def body(cur_ref, sp_ref, ewni_ref, ws_ref, b_ref, o_ref):
        agg = (sp_ref[0] + sp_ref[1]) * ewni_ref[...]
        v = cur_ref[...] + agg
        o_ref[...] = jnp.maximum(v * ws_ref[...][None, :] + b_ref[...][None, :],
                                 0.0)

    return pl.pallas_call(
        body,
        out_shape=jax.ShapeDtypeStruct((NP, H), jnp.float32),
        grid=(_GRID,),
        in_specs=[_row_spec(1), _part_spec(1), _row_spec(1),
                  _full_spec((H,)), _full_spec((H,))],
        out_specs=_row_spec(H),
    )


def kernel(x, edge_index, edge_attr, W_g0, b_g0, W_g1, b_g1, W_g2, b_g2, W_ih, W_hh, b_ih, b_hh, W_h1, b_h1, W_h2, b_h2, W_h3, b_h3, W_fe1, b_fe1, W_fe2, b_fe2, W_fd1, b_fd1, W_fd2, b_fd2, dt):
    f32 = jnp.float32
    srcp = jnp.pad(edge_index[0], (0, EP - E))
    dstp = jnp.pad(edge_index[1], (0, EP - E))
    distp = jnp.pad(edge_attr[:, 0], (0, EP - E), constant_values=1.0)
    ea1p = jnp.pad(edge_attr[:, 1], (0, EP - E))
    pad128 = lambda a: jnp.pad(a, (0, 128))
    spad = pad128(srcp)
    dpad = pad128(dstp)
    srcA1 = pad128(srcp * 2)
    srcB1 = pad128(srcp * 2 + 1)
    x_p = jnp.pad(x, ((0, NP - N), (0, 0), (0, 0)))
    x2d = x.reshape(N * TIN, F)
    dt16 = jnp.full((16,), dt, f32)
    wfd = jnp.concatenate([
        W_fd1.reshape(-1), b_fd1, W_fd2.reshape(-1), b_fd2,
        jnp.zeros((176 - 96 - 32 - 32 - 1,), f32)])
    wsum0 = jnp.sum(W_g0, axis=1)
    Wg0t = W_g0.T
    Wg1t = W_g1.T
    Wg2t = W_g2.T
    Wiht = W_ih.T
    Whht = W_hh.T
    Wh1t = W_h1.T
    Wh2t = W_h2.T
    Wh3t = W_h3.T
    Wfe1t = W_fe1.T
    Wfe2t = W_fe2.T

    o_gs, ew_e, c_e = _graph_static_kernel()(srcp, dstp, distp, ea1p)
    ewpad = pad128(ew_e)
    o_gs = o_gs.reshape(NC, 6, NP)
    down, dx8, ewni, bcolf, ivxf, degi = _tc_finalize()(o_gs)
    ewni_c = ewni.reshape(NP, 1)
    bcol_c = bcolf.reshape(NP, 1)
    ivx_c = ivxf.reshape(NP, 1)
    bcol_r = bcolf.reshape(NR, 128)
    ivx_r = ivxf.reshape(NR, 128)
    degi_r = degi.reshape(NR, 128)
    down_r = down.reshape(NR, 128)

    xg8 = _gather_x8_kernel()(x2d, dx8.reshape(-1))

    seg_h = _segsum_kernel(NP)
    seg_x = _segsum_kernel(2 * NP)
    post128 = _tc_post128()
    post64 = _tc_post64()
    fuse_enc = _tc_fuse_enc()
    post_gru = _tc_post_gru()
    post_gru_head = _tc_post_gru_head()
    dec_l0 = _dec_l0_kernel()
    dec_l0_post = _tc_dec_l0_post()
    phys_a = _phys_a_kernel()
    phys_b = _phys_b_kernel()

    h = jnp.zeros((NP, H), f32)
    xf = None
    for t in range(TIN):
        xt = x_p[:, t, :]
        xf = fuse_enc(xt, xg8[t], ivx_c, bcol_c, Wfe1t, b_fe1, Wfe2t, b_fe2)
        xf2 = xf.reshape(2 * NP, H)
        pa = seg_x(xf2, srcA1, dpad, ewpad)
        pb = seg_x(xf2, srcB1, dpad, ewpad)
        h1 = post128(xf, pa, pb, ewni_c, Wg0t, b_g0)
        p1 = seg_h(h1, spad, dpad, ewpad)
        h2 = post64(h1, p1, ewni_c, Wg1t, b_g1)
        p2 = seg_h(h2, spad, dpad, ewpad)
        h = post_gru(h2, p2, ewni_c, Wg2t, b_g2, h, Wiht, Whht, b_ih, b_hh)

    vbase = xf[:, 0].reshape(NR, 128)
    parts = jnp.zeros((NC, 2, NR, 128), f32)
    preds = []
    for t in range(TOUT):
        cur, sp = dec_l0(vbase, parts, degi_r, dt16, ew_e, srcp, dstp)
        h1 = dec_l0_post(cur.reshape(NP, 1), sp.reshape(NC, NP, 1), ewni_c,
                         wsum0, b_g0)
        p1 = seg_h(h1, spad, dpad, ewpad)
        h2 = post64(h1, p1, ewni_c, Wg1t, b_g1)
        p2 = seg_h(h2, spad, dpad, ewpad)
        h, y = post_gru_head(h2, p2, ewni_c, Wg2t, b_g2, h,
                             Wiht, Whht, b_ih, b_hh,
                             Wh1t, b_h1, Wh2t, b_h2, Wh3t, b_h3)
        preds.append(y[:N])
        if t < TOUT - 1:
            nxt0, partsA = phys_a(y.reshape(NR, 128), down_r, ivx_r, bcol_r,
                                  wfd, ew_e, c_e, srcp, dstp)
            vbase, parts = phys_b(nxt0, partsA, bcol_r, degi_r, dt16,
                                  ew_e, c_e, srcp, dstp)
    return jnp.concatenate(preds, axis=1)
